# Initial kernel scaffold; baseline (speedup 1.0000x reference)
import jax
import jax.numpy as jnp
from jax.experimental import pallas as pl


def kernel(x, edge_index, edge_attr, batch, W0, att_src0, att_dst0, att_edge0, We0, b0, W1, att_src1, att_dst1, att_edge1, We1, b1, gate_w, gate_b, lin1_w, lin1_b):
    raise NotImplementedError("write your pallas kernel here")


# speedup vs baseline: 24.7924x; 24.7924x over previous
import functools

import jax
import jax.numpy as jnp
from jax import lax
from jax.experimental import pallas as pl
from jax.experimental.pallas import tpu as pltpu
from jax.experimental.pallas import tpu_sc as plsc

N = 10000
E = 320000
HID = 128
G = 64

NC = 2
NS = 16
LANES = 16
NW = NC * NS
EPW = E // NW
SEG = 2000
NSEG = EPW // SEG
CHUNK = 80
NCHUNK = SEG // CHUNK
RPS = N // NS

_f32 = jnp.float32


def _prep_body(x_ref, w_ref, as_ref, ad_ref, h_ref, ss_ref, sd_ref):
    h = jnp.dot(x_ref[...], w_ref[...], preferred_element_type=_f32)
    h_ref[...] = h
    ss_ref[...] = jnp.sum(h * as_ref[...][None, :], axis=1)
    sd_ref[...] = jnp.sum(h * ad_ref[...][None, :], axis=1)


def _prep_tc(x, W, a_s, a_d):
    return pl.pallas_call(
        _prep_body,
        out_shape=(
            jax.ShapeDtypeStruct((N, HID), _f32),
            jax.ShapeDtypeStruct((N,), _f32),
            jax.ShapeDtypeStruct((N,), _f32),
        ),
    )(x, W, a_s, a_d)


_EPR = 8
_ED = 16


def _escore_body(ea_ref, we0_ref, ae0_ref, we1_ref, ae1_ref, out_ref):
    def tiled_w(we_ref, ae_ref):
        t = jnp.concatenate([we_ref[...]] * _EPR, axis=0)
        return jnp.sum(t * ae_ref[...][None, :], axis=1)

    wt0 = tiled_w(we0_ref, ae0_ref)
    wt1 = tiled_w(we1_ref, ae1_ref)
    gid = lax.broadcasted_iota(jnp.int32, (HID, _EPR), 0) // _ED
    cid = lax.broadcasted_iota(jnp.int32, (HID, _EPR), 1)
    c = (gid == cid).astype(_f32)
    b = jnp.concatenate([c * wt0[:, None], c * wt1[:, None]], axis=1)
    out_ref[...] = jnp.dot(ea_ref[...], b, preferred_element_type=_f32)


def _escore_tc(ea8, We0, ae0, We1, ae1):
    nblk = 8
    blk = ea8.shape[0] // nblk
    return pl.pallas_call(
        _escore_body,
        grid=(nblk,),
        in_specs=[
            pl.BlockSpec((blk, HID), lambda i: (i, 0)),
            pl.BlockSpec(We0.shape, lambda i: (0, 0)),
            pl.BlockSpec(ae0.shape, lambda i: (0,)),
            pl.BlockSpec(We1.shape, lambda i: (0, 0)),
            pl.BlockSpec(ae1.shape, lambda i: (0,)),
        ],
        out_specs=pl.BlockSpec((blk, 2 * _EPR), lambda i: (i, 0)),
        out_shape=jax.ShapeDtypeStruct((ea8.shape[0], 2 * _EPR), _f32),
    )(ea8, We0, ae0, We1, ae1)


_SC_PARAMS = pltpu.CompilerParams(use_tc_tiling_on_sc=False,
                                  needs_layout_passes=False)


def _sc_mesh():
    return plsc.VectorSubcoreMesh(core_axis_name="c", subcore_axis_name="s",
                                  num_cores=NC, num_subcores=NS)


def _sc_deg_body(dst_hbm, esc0_hbm, esc1_hbm,
                 cnt_hbm, esum0_hbm, esum1_hbm,
                 dst_v, esc0_v, esc1_v, cnt_v, esum0_v, esum1_v):
    cid = lax.axis_index("c")
    sid = lax.axis_index("s")
    wid = sid * NC + cid

    pltpu.sync_copy(dst_hbm.at[wid], dst_v)
    pltpu.sync_copy(esc0_hbm.at[wid], esc0_v)
    pltpu.sync_copy(esc1_hbm.at[wid], esc1_v)

    zeros16 = jnp.zeros((LANES,), _f32)

    @pl.loop(0, N // LANES)
    def _(i):
        sl = pl.ds(i * LANES, LANES)
        cnt_v[sl] = zeros16
        esum0_v[sl] = zeros16
        esum1_v[sl] = zeros16

    ones16 = jnp.ones((LANES,), _f32)

    @pl.loop(0, EPW // LANES)
    def _(g):
        sl = pl.ds(g * LANES, LANES)
        dsti = dst_v[sl]
        plsc.addupdate_scatter(cnt_v, [dsti], ones16)
        plsc.addupdate_scatter(esum0_v, [dsti], esc0_v[sl])
        plsc.addupdate_scatter(esum1_v, [dsti], esc1_v[sl])

    pltpu.sync_copy(cnt_v, cnt_hbm.at[wid])
    pltpu.sync_copy(esum0_v, esum0_hbm.at[wid])
    pltpu.sync_copy(esum1_v, esum1_hbm.at[wid])


def _sc_deg_pass(dst2d, esc0, esc1):
    f = pl.kernel(
        _sc_deg_body,
        out_type=(
            jax.ShapeDtypeStruct((NW, N), _f32),
            jax.ShapeDtypeStruct((NW, N), _f32),
            jax.ShapeDtypeStruct((NW, N), _f32),
        ),
        mesh=_sc_mesh(),
        compiler_params=_SC_PARAMS,
        scratch_types=[
            pltpu.VMEM((EPW,), jnp.int32),
            pltpu.VMEM((EPW,), _f32),
            pltpu.VMEM((EPW,), _f32),
            pltpu.VMEM((N,), _f32),
            pltpu.VMEM((N,), _f32),
            pltpu.VMEM((N,), _f32),
        ],
    )
    return f(dst2d, esc0, esc1)


def _sc_edge_body(h_hbm, src_hbm, dst_hbm, esc_hbm, ssrc_hbm, sdst_hbm,
                  num_hbm, den_hbm,
                  ssrc_v, sdst_v, den_v, src_v, dst_v, esc_v, ex_v,
                  rows_v, acc_sh, sem):
    cid = lax.axis_index("c")
    sid = lax.axis_index("s")
    wid = sid * NC + cid

    pltpu.sync_copy(ssrc_hbm, ssrc_v)
    pltpu.sync_copy(sdst_hbm, sdst_v)

    zeros16 = jnp.zeros((LANES,), _f32)

    @pl.loop(0, N // LANES)
    def _(i):
        den_v[pl.ds(i * LANES, LANES)] = zeros16

    @pl.loop(0, CHUNK)
    def _(r):
        for k in range(HID // LANES):
            rows_v[r, pl.ds(k * LANES, LANES)] = zeros16

    @pl.loop(0, RPS // CHUNK)
    def _(z):
        pltpu.sync_copy(rows_v,
                        acc_sh.at[pl.ds(sid * RPS + z * CHUNK, CHUNK)])
    rem = RPS - (RPS // CHUNK) * CHUNK
    pltpu.sync_copy(rows_v.at[pl.ds(0, rem)],
                    acc_sh.at[pl.ds(sid * RPS + RPS - rem, rem)])
    plsc.subcore_barrier()

    @pl.loop(0, NSEG)
    def _(s):
        pltpu.sync_copy(src_hbm.at[wid].at[s], src_v)
        pltpu.sync_copy(dst_hbm.at[wid].at[s], dst_v)
        pltpu.sync_copy(esc_hbm.at[wid].at[s], esc_v)

        @pl.loop(0, NCHUNK)
        def _(j):
            off = j * CHUNK
            pltpu.async_copy(h_hbm.at[src_v.at[pl.ds(off, CHUNK)]],
                             rows_v, sem).wait()

            for g in range(CHUNK // LANES):
                sl = pl.ds(off + g * LANES, LANES)
                srci = src_v[sl]
                dsti = dst_v[j, pl.ds(g * LANES, LANES)]
                a = (plsc.load_gather(ssrc_v, [srci])
                     + plsc.load_gather(sdst_v, [dsti]) + esc_v[sl])
                a = jnp.where(a > 0.0, a, 0.2 * a)
                ex = jnp.exp(a)
                ex_v[pl.ds(g * LANES, LANES)] = ex
                plsc.addupdate_scatter(den_v, [dsti], ex)

            @pl.loop(0, CHUNK)
            def _(r):
                exs = ex_v[pl.ds(r, LANES)][0]
                for k in range(HID // LANES):
                    rsl = pl.ds(k * LANES, LANES)
                    rows_v[r, rsl] = rows_v[r, rsl] * exs

            pltpu.sync_copy(rows_v, acc_sh.at[dst_v.at[j]], add=True)

    plsc.subcore_barrier()

    pltpu.sync_copy(acc_sh.at[pl.ds(sid * RPS, RPS)],
                    num_hbm.at[cid].at[pl.ds(sid * RPS, RPS)])
    pltpu.sync_copy(den_v, den_hbm.at[wid])


def _sc_edge_pass(h, src3d, dst4d, esc3d, ssrc, sdst):
    f = pl.kernel(
        _sc_edge_body,
        out_type=(
            jax.ShapeDtypeStruct((NC, N, HID), _f32),
            jax.ShapeDtypeStruct((NW, N), _f32),
        ),
        mesh=_sc_mesh(),
        compiler_params=_SC_PARAMS,
        scratch_types=[
            pltpu.VMEM((N,), _f32),
            pltpu.VMEM((N,), _f32),
            pltpu.VMEM((N,), _f32),
            pltpu.VMEM((SEG,), jnp.int32),
            pltpu.VMEM((NCHUNK, CHUNK), jnp.int32),
            pltpu.VMEM((SEG,), _f32),
            pltpu.VMEM((CHUNK + LANES,), _f32),
            pltpu.VMEM((CHUNK, HID), _f32),
            pltpu.VMEM_SHARED((N, HID), _f32),
            pltpu.SemaphoreType.DMA,
        ],
    )
    return f(h, src3d, dst4d, esc3d, ssrc, sdst)


def _mid_body(nump_ref, denp_ref, cntp_ref, esump_ref, h0_ref, ss0_ref,
              sd0_ref, b0_ref, w1_ref, as1_ref, ad1_ref,
              h1_ref, ss1_ref, sd1_ref, cnt_ref):
    den = jnp.sum(denp_ref[...], axis=0)
    cnt = jnp.sum(cntp_ref[...], axis=0)
    esum = jnp.sum(esump_ref[...], axis=0)
    num = nump_ref[0] + nump_ref[1]
    a_self = ss0_ref[...] + sd0_ref[...] + esum / jnp.maximum(cnt, 1.0)
    a_self = jnp.where(a_self > 0.0, a_self, 0.2 * a_self)
    exs = jnp.exp(a_self)
    h0 = h0_ref[...]
    num = num + exs[:, None] * h0
    den = den + exs
    x1 = num / (den + 1e-16)[:, None] + b0_ref[...][None, :]
    x1 = jnp.maximum(x1, 0.0)
    h1 = jnp.dot(x1, w1_ref[...], preferred_element_type=_f32)
    h1_ref[...] = h1
    ss1_ref[...] = jnp.sum(h1 * as1_ref[...][None, :], axis=1)
    sd1_ref[...] = jnp.sum(h1 * ad1_ref[...][None, :], axis=1)
    cnt_ref[...] = cnt


def _mid_tc(num_p, den_p, cnt_p, esum_p, h0, ss0, sd0, b0, W1, as1, ad1):
    return pl.pallas_call(
        _mid_body,
        out_shape=(
            jax.ShapeDtypeStruct((N, HID), _f32),
            jax.ShapeDtypeStruct((N,), _f32),
            jax.ShapeDtypeStruct((N,), _f32),
            jax.ShapeDtypeStruct((N,), _f32),
        ),
    )(num_p, den_p, cnt_p, esum_p, h0, ss0, sd0, b0, W1, as1, ad1)


def _final_body(nump_ref, denp_ref, esump_ref, cnt_ref, h1_ref, ss1_ref,
                sd1_ref, b1_ref, batch_ref, gw_ref, gb_ref, lw_ref, lb_ref,
                out_ref):
    den = jnp.sum(denp_ref[...], axis=0)
    esum = jnp.sum(esump_ref[...], axis=0)
    num = nump_ref[0] + nump_ref[1]
    a_self = ss1_ref[...] + sd1_ref[...] + esum / jnp.maximum(cnt_ref[...], 1.0)
    a_self = jnp.where(a_self > 0.0, a_self, 0.2 * a_self)
    exs = jnp.exp(a_self)
    h1 = h1_ref[...]
    num = num + exs[:, None] * h1
    den = den + exs
    h2 = num / (den + 1e-16)[:, None] + b1_ref[...][None, :]

    gate = jnp.sum(h2 * gw_ref[...][:, 0][None, :], axis=1) + gb_ref[0]
    ids = lax.broadcasted_iota(jnp.int32, (G, N), 0)
    msk = batch_ref[...][None, :] == ids
    m = jnp.max(jnp.where(msk, gate[None, :], -1e30), axis=1)
    mb = jnp.sum(jnp.where(msk, m[:, None], 0.0), axis=0)
    ex = jnp.exp(gate - mb)
    den_g = jnp.sum(jnp.where(msk, ex[None, :], 0.0), axis=1)
    denb = jnp.sum(jnp.where(msk, den_g[:, None], 0.0), axis=0)
    coef = ex / (denb + 1e-16)
    pool_w = jnp.where(msk, coef[None, :], 0.0)
    pooled = jnp.dot(pool_w, h2, preferred_element_type=_f32)
    out_ref[...] = (jnp.dot(pooled, lw_ref[...], preferred_element_type=_f32)
                    + lb_ref[...][None, :])


def _final_tc(num_p, den_p, esum_p, cnt, h1, ss1, sd1, b1, batch, gw, gb,
              lw, lb):
    return pl.pallas_call(
        _final_body,
        out_shape=jax.ShapeDtypeStruct((G, HID), _f32),
    )(num_p, den_p, esum_p, cnt, h1, ss1, sd1, b1, batch, gw, gb, lw, lb)


def kernel(x, edge_index, edge_attr, batch, W0, att_src0, att_dst0,
           att_edge0, We0, b0, W1, att_src1, att_dst1, att_edge1, We1, b1,
           gate_w, gate_b, lin1_w, lin1_b):
    src3d = edge_index[0].reshape(NW, NSEG, SEG)
    dst2d = edge_index[1].reshape(NW, EPW)
    dst4d = edge_index[1].reshape(NW, NSEG, NCHUNK, CHUNK)

    h0, ss0, sd0 = _prep_tc(x, W0, att_src0, att_dst0)
    ea8 = edge_attr.reshape(E // _EPR, HID)
    escp = _escore_tc(ea8, We0, att_edge0, We1, att_edge1)
    esc0 = escp[:, :_EPR].reshape(E)
    esc1 = escp[:, _EPR:].reshape(E)

    cnt_p, esum0_p, esum1_p = _sc_deg_pass(
        dst2d, esc0.reshape(NW, EPW), esc1.reshape(NW, EPW))
    num0, den0 = _sc_edge_pass(
        h0, src3d, dst4d, esc0.reshape(NW, NSEG, SEG), ss0, sd0)
    h1, ss1, sd1, cnt = _mid_tc(num0, den0, cnt_p, esum0_p, h0, ss0, sd0, b0,
                                W1, att_src1, att_dst1)
    num1, den1 = _sc_edge_pass(
        h1, src3d, dst4d, esc1.reshape(NW, NSEG, SEG), ss1, sd1)
    return _final_tc(num1, den1, esum1_p, cnt, h1, ss1, sd1, b1, batch,
                     gate_w, gate_b, lin1_w, lin1_b)


# speedup vs baseline: 34.4489x; 1.3895x over previous
import functools

import jax
import jax.numpy as jnp
from jax import lax
from jax.experimental import pallas as pl
from jax.experimental.pallas import tpu as pltpu
from jax.experimental.pallas import tpu_sc as plsc

N = 10000
E = 320000
HID = 128
G = 64

NC = 2
NS = 16
LANES = 16
NW = NC * NS
EPW = E // NW
SEG = 2000
NSEG = EPW // SEG
CHUNK = 80
NCHUNK = SEG // CHUNK
RPS = N // NS

_f32 = jnp.float32


def _prep_body(x_ref, w_ref, as_ref, ad_ref, h_ref, ss_ref, sd_ref):
    h = jnp.dot(x_ref[...], w_ref[...], preferred_element_type=_f32)
    h_ref[...] = h
    ss_ref[...] = jnp.sum(h * as_ref[...][None, :], axis=1)
    sd_ref[...] = jnp.sum(h * ad_ref[...][None, :], axis=1)


def _prep_tc(x, W, a_s, a_d):
    return pl.pallas_call(
        _prep_body,
        out_shape=(
            jax.ShapeDtypeStruct((N, HID), _f32),
            jax.ShapeDtypeStruct((N,), _f32),
            jax.ShapeDtypeStruct((N,), _f32),
        ),
    )(x, W, a_s, a_d)


_EPR = 8
_ED = 16


def _escore_body(ea_ref, we0_ref, ae0_ref, we1_ref, ae1_ref, e0_ref, e1_ref):
    def tiled_w(we_ref, ae_ref):
        t = jnp.concatenate([we_ref[...]] * _EPR, axis=0)
        return jnp.sum(t * ae_ref[...][None, :], axis=1)

    wt0 = tiled_w(we0_ref, ae0_ref)
    wt1 = tiled_w(we1_ref, ae1_ref)
    gid = lax.broadcasted_iota(jnp.int32, (HID, _EPR), 0) // _ED
    cid = lax.broadcasted_iota(jnp.int32, (HID, _EPR), 1)
    c = (gid == cid).astype(_f32)
    b = jnp.concatenate([c * wt0[:, None], c * wt1[:, None]], axis=1)
    res = jnp.dot(ea_ref[...], b, preferred_element_type=_f32)
    e0_ref[...] = res[:, :_EPR]
    e1_ref[...] = res[:, _EPR:]


def _escore_tc(ea8, We0, ae0, We1, ae1):
    nblk = 8
    blk = ea8.shape[0] // nblk
    return pl.pallas_call(
        _escore_body,
        grid=(nblk,),
        in_specs=[
            pl.BlockSpec((blk, HID), lambda i: (i, 0)),
            pl.BlockSpec(We0.shape, lambda i: (0, 0)),
            pl.BlockSpec(ae0.shape, lambda i: (0,)),
            pl.BlockSpec(We1.shape, lambda i: (0, 0)),
            pl.BlockSpec(ae1.shape, lambda i: (0,)),
        ],
        out_specs=(
            pl.BlockSpec((blk, _EPR), lambda i: (i, 0)),
            pl.BlockSpec((blk, _EPR), lambda i: (i, 0)),
        ),
        out_shape=(
            jax.ShapeDtypeStruct((ea8.shape[0], _EPR), _f32),
            jax.ShapeDtypeStruct((ea8.shape[0], _EPR), _f32),
        ),
    )(ea8, We0, ae0, We1, ae1)


_SC_PARAMS = pltpu.CompilerParams(use_tc_tiling_on_sc=False,
                                  needs_layout_passes=False)


def _sc_mesh():
    return plsc.VectorSubcoreMesh(core_axis_name="c", subcore_axis_name="s",
                                  num_cores=NC, num_subcores=NS)


def _sc_deg_body(dst_hbm, esc0_hbm, esc1_hbm,
                 cnt_hbm, esum0_hbm, esum1_hbm,
                 dst_v, esc0_v, esc1_v, cnt_v, esum0_v, esum1_v):
    cid = lax.axis_index("c")
    sid = lax.axis_index("s")
    wid = sid * NC + cid

    pltpu.sync_copy(dst_hbm.at[wid], dst_v)
    pltpu.sync_copy(esc0_hbm.at[wid], esc0_v)
    pltpu.sync_copy(esc1_hbm.at[wid], esc1_v)

    zeros16 = jnp.zeros((LANES,), _f32)

    @pl.loop(0, N // LANES)
    def _(i):
        sl = pl.ds(i * LANES, LANES)
        cnt_v[sl] = zeros16
        esum0_v[sl] = zeros16
        esum1_v[sl] = zeros16

    ones16 = jnp.ones((LANES,), _f32)

    @pl.loop(0, EPW // LANES)
    def _(g):
        sl = pl.ds(g * LANES, LANES)
        dsti = dst_v[sl]
        plsc.addupdate_scatter(cnt_v, [dsti], ones16)
        plsc.addupdate_scatter(esum0_v, [dsti], esc0_v[sl])
        plsc.addupdate_scatter(esum1_v, [dsti], esc1_v[sl])

    pltpu.sync_copy(cnt_v, cnt_hbm.at[wid])
    pltpu.sync_copy(esum0_v, esum0_hbm.at[wid])
    pltpu.sync_copy(esum1_v, esum1_hbm.at[wid])


def _sc_deg_pass(dst2d, esc0, esc1):
    f = pl.kernel(
        _sc_deg_body,
        out_type=(
            jax.ShapeDtypeStruct((NW, N), _f32),
            jax.ShapeDtypeStruct((NW, N), _f32),
            jax.ShapeDtypeStruct((NW, N), _f32),
        ),
        mesh=_sc_mesh(),
        compiler_params=_SC_PARAMS,
        scratch_types=[
            pltpu.VMEM((EPW,), jnp.int32),
            pltpu.VMEM((EPW,), _f32),
            pltpu.VMEM((EPW,), _f32),
            pltpu.VMEM((N,), _f32),
            pltpu.VMEM((N,), _f32),
            pltpu.VMEM((N,), _f32),
        ],
    )
    return f(dst2d, esc0, esc1)


def _sc_scalar_body(dst_hbm, val_hbm, out_hbm, dst_v, val_v, acc_v):
    cid = lax.axis_index("c")
    sid = lax.axis_index("s")
    wid = sid * NC + cid

    pltpu.sync_copy(dst_hbm.at[wid], dst_v)
    pltpu.sync_copy(val_hbm.at[wid], val_v)

    zeros16 = jnp.zeros((LANES,), _f32)

    @pl.loop(0, N // LANES)
    def _(i):
        acc_v[pl.ds(i * LANES, LANES)] = zeros16

    @pl.loop(0, EPW // LANES)
    def _(g):
        sl = pl.ds(g * LANES, LANES)
        plsc.addupdate_scatter(acc_v, [dst_v[sl]], val_v[sl])

    pltpu.sync_copy(acc_v, out_hbm.at[wid])


def _sc_scalar_pass(dst2d, val2d):
    f = pl.kernel(
        _sc_scalar_body,
        out_type=jax.ShapeDtypeStruct((NW, N), _f32),
        mesh=_sc_mesh(),
        compiler_params=_SC_PARAMS,
        scratch_types=[
            pltpu.VMEM((EPW,), jnp.int32),
            pltpu.VMEM((EPW,), _f32),
            pltpu.VMEM((N,), _f32),
        ],
    )
    return f(dst2d, val2d)


def _sc_edge_body(h_hbm, src_hbm, dst_hbm, esc_hbm, ssrc_hbm, sdst_hbm,
                  num_hbm, ex_hbm,
                  ssrc_v, sdst_v, src_v, dst_v, esc_v, ex_v,
                  rows_a, rows_b, acc_sh, sem_a, sem_b):
    cid = lax.axis_index("c")
    sid = lax.axis_index("s")
    wid = sid * NC + cid

    pltpu.sync_copy(ssrc_hbm, ssrc_v)
    pltpu.sync_copy(sdst_hbm, sdst_v)

    zeros16 = jnp.zeros((LANES,), _f32)

    @pl.loop(0, CHUNK)
    def _(r):
        for k in range(HID // LANES):
            rows_a[r, pl.ds(k * LANES, LANES)] = zeros16

    @pl.loop(0, RPS // CHUNK)
    def _(z):
        pltpu.sync_copy(rows_a,
                        acc_sh.at[pl.ds(sid * RPS + z * CHUNK, CHUNK)])
    rem = RPS - (RPS // CHUNK) * CHUNK
    pltpu.sync_copy(rows_a.at[pl.ds(0, rem)],
                    acc_sh.at[pl.ds(sid * RPS + RPS - rem, rem)])
    plsc.subcore_barrier()

    def start_gather(j, buf, sem):
        return pltpu.async_copy(
            h_hbm.at[src_v.at[pl.ds(j * CHUNK, CHUNK)]], buf, sem)

    def wait_gather(j, buf, sem):
        pltpu.make_async_copy(
            h_hbm.at[src_v.at[pl.ds(j * CHUNK, CHUNK)]], buf, sem).wait()

    def scalars(j):
        off = j * CHUNK
        for g in range(CHUNK // LANES):
            sl = pl.ds(off + g * LANES, LANES)
            srci = src_v[sl]
            dsti = dst_v[j, pl.ds(g * LANES, LANES)]
            a = (plsc.load_gather(ssrc_v, [srci])
                 + plsc.load_gather(sdst_v, [dsti]) + esc_v[sl])
            a = jnp.where(a > 0.0, a, 0.2 * a)
            ex_v[sl] = jnp.exp(a)

    def scale_scatter(j, buf):
        off = j * CHUNK

        @pl.loop(0, CHUNK)
        def _(r):
            exs = ex_v[pl.ds(off + r, LANES)][0]
            for k in range(HID // LANES):
                rsl = pl.ds(k * LANES, LANES)
                buf[r, rsl] = buf[r, rsl] * exs

        pltpu.sync_copy(buf, acc_sh.at[dst_v.at[j]], add=True)

    @pl.loop(0, NSEG)
    def _(s):
        pltpu.sync_copy(src_hbm.at[wid].at[s], src_v)
        pltpu.sync_copy(dst_hbm.at[wid].at[s], dst_v)
        pltpu.sync_copy(esc_hbm.at[wid].at[s], esc_v)

        start_gather(0, rows_a, sem_a)

        @pl.loop(0, (NCHUNK - 1) // 2)
        def _(t):
            j0 = 2 * t
            start_gather(j0 + 1, rows_b, sem_b)
            scalars(j0)
            wait_gather(j0, rows_a, sem_a)
            scale_scatter(j0, rows_a)
            start_gather(j0 + 2, rows_a, sem_a)
            scalars(j0 + 1)
            wait_gather(j0 + 1, rows_b, sem_b)
            scale_scatter(j0 + 1, rows_b)

        scalars(NCHUNK - 1)
        wait_gather(NCHUNK - 1, rows_a, sem_a)
        scale_scatter(NCHUNK - 1, rows_a)

        pltpu.sync_copy(ex_v.at[pl.ds(0, SEG)], ex_hbm.at[wid].at[s])

    plsc.subcore_barrier()

    pltpu.sync_copy(acc_sh.at[pl.ds(sid * RPS, RPS)],
                    num_hbm.at[cid].at[pl.ds(sid * RPS, RPS)])


def _sc_edge_pass(h, src3d, dst4d, esc3d, ssrc, sdst):
    f = pl.kernel(
        _sc_edge_body,
        out_type=(
            jax.ShapeDtypeStruct((NC, N, HID), _f32),
            jax.ShapeDtypeStruct((NW, NSEG, SEG), _f32),
        ),
        mesh=_sc_mesh(),
        compiler_params=_SC_PARAMS,
        scratch_types=[
            pltpu.VMEM((N,), _f32),
            pltpu.VMEM((N,), _f32),
            pltpu.VMEM((SEG,), jnp.int32),
            pltpu.VMEM((NCHUNK, CHUNK), jnp.int32),
            pltpu.VMEM((SEG,), _f32),
            pltpu.VMEM((SEG + LANES,), _f32),
            pltpu.VMEM((CHUNK, HID), _f32),
            pltpu.VMEM((CHUNK, HID), _f32),
            pltpu.VMEM_SHARED((N, HID), _f32),
            pltpu.SemaphoreType.DMA,
            pltpu.SemaphoreType.DMA,
        ],
    )
    return f(h, src3d, dst4d, esc3d, ssrc, sdst)


def _mid_body(nump_ref, denp_ref, cntp_ref, esump_ref, h0_ref, ss0_ref,
              sd0_ref, b0_ref, w1_ref, as1_ref, ad1_ref,
              h1_ref, ss1_ref, sd1_ref, cnt_ref):
    den = jnp.sum(denp_ref[...], axis=0)
    cnt = jnp.sum(cntp_ref[...], axis=0)
    esum = jnp.sum(esump_ref[...], axis=0)
    num = nump_ref[0] + nump_ref[1]
    a_self = ss0_ref[...] + sd0_ref[...] + esum / jnp.maximum(cnt, 1.0)
    a_self = jnp.where(a_self > 0.0, a_self, 0.2 * a_self)
    exs = jnp.exp(a_self)
    h0 = h0_ref[...]
    num = num + exs[:, None] * h0
    den = den + exs
    x1 = num / (den + 1e-16)[:, None] + b0_ref[...][None, :]
    x1 = jnp.maximum(x1, 0.0)
    h1 = jnp.dot(x1, w1_ref[...], preferred_element_type=_f32)
    h1_ref[...] = h1
    ss1_ref[...] = jnp.sum(h1 * as1_ref[...][None, :], axis=1)
    sd1_ref[...] = jnp.sum(h1 * ad1_ref[...][None, :], axis=1)
    cnt_ref[...] = cnt


def _mid_tc(num_p, den_p, cnt_p, esum_p, h0, ss0, sd0, b0, W1, as1, ad1):
    return pl.pallas_call(
        _mid_body,
        out_shape=(
            jax.ShapeDtypeStruct((N, HID), _f32),
            jax.ShapeDtypeStruct((N,), _f32),
            jax.ShapeDtypeStruct((N,), _f32),
            jax.ShapeDtypeStruct((N,), _f32),
        ),
    )(num_p, den_p, cnt_p, esum_p, h0, ss0, sd0, b0, W1, as1, ad1)


def _final_body(nump_ref, denp_ref, esump_ref, cnt_ref, h1_ref, ss1_ref,
                sd1_ref, b1_ref, batch_ref, gw_ref, gb_ref, lw_ref, lb_ref,
                out_ref):
    den = jnp.sum(denp_ref[...], axis=0)
    esum = jnp.sum(esump_ref[...], axis=0)
    num = nump_ref[0] + nump_ref[1]
    a_self = ss1_ref[...] + sd1_ref[...] + esum / jnp.maximum(cnt_ref[...], 1.0)
    a_self = jnp.where(a_self > 0.0, a_self, 0.2 * a_self)
    exs = jnp.exp(a_self)
    h1 = h1_ref[...]
    num = num + exs[:, None] * h1
    den = den + exs
    h2 = num / (den + 1e-16)[:, None] + b1_ref[...][None, :]

    gate = jnp.sum(h2 * gw_ref[...][:, 0][None, :], axis=1) + gb_ref[0]
    ids = lax.broadcasted_iota(jnp.int32, (G, N), 0)
    msk = batch_ref[...][None, :] == ids
    m = jnp.max(jnp.where(msk, gate[None, :], -1e30), axis=1)
    mb = jnp.sum(jnp.where(msk, m[:, None], 0.0), axis=0)
    ex = jnp.exp(gate - mb)
    den_g = jnp.sum(jnp.where(msk, ex[None, :], 0.0), axis=1)
    denb = jnp.sum(jnp.where(msk, den_g[:, None], 0.0), axis=0)
    coef = ex / (denb + 1e-16)
    pool_w = jnp.where(msk, coef[None, :], 0.0)
    pooled = jnp.dot(pool_w, h2, preferred_element_type=_f32)
    out_ref[...] = (jnp.dot(pooled, lw_ref[...], preferred_element_type=_f32)
                    + lb_ref[...][None, :])


def _final_tc(num_p, den_p, esum_p, cnt, h1, ss1, sd1, b1, batch, gw, gb,
              lw, lb):
    return pl.pallas_call(
        _final_body,
        out_shape=jax.ShapeDtypeStruct((G, HID), _f32),
    )(num_p, den_p, esum_p, cnt, h1, ss1, sd1, b1, batch, gw, gb, lw, lb)


def kernel(x, edge_index, edge_attr, batch, W0, att_src0, att_dst0,
           att_edge0, We0, b0, W1, att_src1, att_dst1, att_edge1, We1, b1,
           gate_w, gate_b, lin1_w, lin1_b):
    src3d = edge_index[0].reshape(NW, NSEG, SEG)
    dst2d = edge_index[1].reshape(NW, EPW)
    dst4d = edge_index[1].reshape(NW, NSEG, NCHUNK, CHUNK)

    h0, ss0, sd0 = _prep_tc(x, W0, att_src0, att_dst0)
    ea8 = edge_attr.reshape(E // _EPR, HID)
    e0p, e1p = _escore_tc(ea8, We0, att_edge0, We1, att_edge1)
    esc0 = e0p.reshape(E)
    esc1 = e1p.reshape(E)

    cnt_p, esum0_p, esum1_p = _sc_deg_pass(
        dst2d, esc0.reshape(NW, EPW), esc1.reshape(NW, EPW))
    num0, ex0 = _sc_edge_pass(
        h0, src3d, dst4d, esc0.reshape(NW, NSEG, SEG), ss0, sd0)
    den0_p = _sc_scalar_pass(dst2d, ex0.reshape(NW, EPW))
    h1, ss1, sd1, cnt = _mid_tc(num0, den0_p, cnt_p, esum0_p, h0, ss0, sd0,
                                b0, W1, att_src1, att_dst1)
    num1, ex1 = _sc_edge_pass(
        h1, src3d, dst4d, esc1.reshape(NW, NSEG, SEG), ss1, sd1)
    den1_p = _sc_scalar_pass(dst2d, ex1.reshape(NW, EPW))
    return _final_tc(num1, den1_p, esum1_p, cnt, h1, ss1, sd1, b1, batch,
                     gate_w, gate_b, lin1_w, lin1_b)


# speedup vs baseline: 34.7676x; 1.0093x over previous
import functools

import jax
import jax.numpy as jnp
from jax import lax
from jax.experimental import pallas as pl
from jax.experimental.pallas import tpu as pltpu
from jax.experimental.pallas import tpu_sc as plsc

N = 10000
E = 320000
HID = 128
G = 64

NC = 2
NS = 16
LANES = 16
NW = NC * NS
EPW = E // NW
SEG = 2000
NSEG = EPW // SEG
CHUNK = 80
NCHUNK = SEG // CHUNK
RPS = N // NS

_f32 = jnp.float32


def _prep_body(x_ref, w_ref, as_ref, ad_ref, h_ref, ss_ref, sd_ref):
    h = jnp.dot(x_ref[...], w_ref[...], preferred_element_type=_f32)
    h_ref[...] = h
    ss_ref[...] = jnp.sum(h * as_ref[...][None, :], axis=1)
    sd_ref[...] = jnp.sum(h * ad_ref[...][None, :], axis=1)


def _prep_tc(x, W, a_s, a_d):
    return pl.pallas_call(
        _prep_body,
        out_shape=(
            jax.ShapeDtypeStruct((N, HID), _f32),
            jax.ShapeDtypeStruct((N,), _f32),
            jax.ShapeDtypeStruct((N,), _f32),
        ),
    )(x, W, a_s, a_d)


_EPR = 8
_ED = 16


def _escore_body(ea_ref, we0_ref, ae0_ref, we1_ref, ae1_ref, e0_ref, e1_ref):
    def tiled_w(we_ref, ae_ref):
        t = jnp.concatenate([we_ref[...]] * _EPR, axis=0)
        return jnp.sum(t * ae_ref[...][None, :], axis=1)

    wt0 = tiled_w(we0_ref, ae0_ref)
    wt1 = tiled_w(we1_ref, ae1_ref)
    gid = lax.broadcasted_iota(jnp.int32, (HID, _EPR), 0) // _ED
    cid = lax.broadcasted_iota(jnp.int32, (HID, _EPR), 1)
    c = (gid == cid).astype(_f32)
    b = jnp.concatenate([c * wt0[:, None], c * wt1[:, None]], axis=1)
    res = jnp.dot(ea_ref[...], b, preferred_element_type=_f32)
    e0_ref[...] = res[:, :_EPR]
    e1_ref[...] = res[:, _EPR:]


def _escore_tc(ea8, We0, ae0, We1, ae1):
    nblk = 8
    blk = ea8.shape[0] // nblk
    return pl.pallas_call(
        _escore_body,
        grid=(nblk,),
        in_specs=[
            pl.BlockSpec((blk, HID), lambda i: (i, 0)),
            pl.BlockSpec(We0.shape, lambda i: (0, 0)),
            pl.BlockSpec(ae0.shape, lambda i: (0,)),
            pl.BlockSpec(We1.shape, lambda i: (0, 0)),
            pl.BlockSpec(ae1.shape, lambda i: (0,)),
        ],
        out_specs=(
            pl.BlockSpec((blk, _EPR), lambda i: (i, 0)),
            pl.BlockSpec((blk, _EPR), lambda i: (i, 0)),
        ),
        out_shape=(
            jax.ShapeDtypeStruct((ea8.shape[0], _EPR), _f32),
            jax.ShapeDtypeStruct((ea8.shape[0], _EPR), _f32),
        ),
    )(ea8, We0, ae0, We1, ae1)


_SC_PARAMS = pltpu.CompilerParams(use_tc_tiling_on_sc=False,
                                  needs_layout_passes=False)


def _sc_mesh():
    return plsc.VectorSubcoreMesh(core_axis_name="c", subcore_axis_name="s",
                                  num_cores=NC, num_subcores=NS)


def _sc_scalar0_body(src_hbm, dst_hbm, esc0_hbm, esc1_hbm, ssrc_hbm,
                     sdst_hbm, ex_hbm, den_hbm, cnt_hbm, esum0_hbm,
                     esum1_hbm, ssrc_v, sdst_v, src_v, dst_v, esc0_v,
                     esc1_v, ex_v, den_v, cnt_v, esum0_v, esum1_v):
    cid = lax.axis_index("c")
    sid = lax.axis_index("s")
    wid = sid * NC + cid

    pltpu.sync_copy(ssrc_hbm, ssrc_v)
    pltpu.sync_copy(sdst_hbm, sdst_v)
    pltpu.sync_copy(src_hbm.at[wid], src_v)
    pltpu.sync_copy(dst_hbm.at[wid], dst_v)
    pltpu.sync_copy(esc0_hbm.at[wid], esc0_v)
    pltpu.sync_copy(esc1_hbm.at[wid], esc1_v)

    zeros16 = jnp.zeros((LANES,), _f32)

    @pl.loop(0, N // LANES)
    def _(i):
        sl = pl.ds(i * LANES, LANES)
        den_v[sl] = zeros16
        cnt_v[sl] = zeros16
        esum0_v[sl] = zeros16
        esum1_v[sl] = zeros16

    ones16 = jnp.ones((LANES,), _f32)

    @pl.loop(0, EPW // LANES)
    def _(g):
        sl = pl.ds(g * LANES, LANES)
        srci = src_v[sl]
        dsti = dst_v[sl]
        esc = esc0_v[sl]
        a = (plsc.load_gather(ssrc_v, [srci])
             + plsc.load_gather(sdst_v, [dsti]) + esc)
        a = jnp.where(a > 0.0, a, 0.2 * a)
        ex = jnp.exp(a)
        ex_v[sl] = ex
        plsc.addupdate_scatter(den_v, [dsti], ex)
        plsc.addupdate_scatter(cnt_v, [dsti], ones16)
        plsc.addupdate_scatter(esum0_v, [dsti], esc)
        plsc.addupdate_scatter(esum1_v, [dsti], esc1_v[sl])

    pltpu.sync_copy(ex_v, ex_hbm.at[wid])
    pltpu.sync_copy(den_v, den_hbm.at[wid])
    pltpu.sync_copy(cnt_v, cnt_hbm.at[wid])
    pltpu.sync_copy(esum0_v, esum0_hbm.at[wid])
    pltpu.sync_copy(esum1_v, esum1_hbm.at[wid])


def _sc_scalar0_pass(src2d, dst2d, esc0, esc1, ssrc, sdst):
    f = pl.kernel(
        _sc_scalar0_body,
        out_type=(
            jax.ShapeDtypeStruct((NW, EPW), _f32),
            jax.ShapeDtypeStruct((NW, N), _f32),
            jax.ShapeDtypeStruct((NW, N), _f32),
            jax.ShapeDtypeStruct((NW, N), _f32),
            jax.ShapeDtypeStruct((NW, N), _f32),
        ),
        mesh=_sc_mesh(),
        compiler_params=_SC_PARAMS,
        scratch_types=[
            pltpu.VMEM((N,), _f32),
            pltpu.VMEM((N,), _f32),
            pltpu.VMEM((EPW,), jnp.int32),
            pltpu.VMEM((EPW,), jnp.int32),
            pltpu.VMEM((EPW,), _f32),
            pltpu.VMEM((EPW,), _f32),
            pltpu.VMEM((EPW,), _f32),
            pltpu.VMEM((N,), _f32),
            pltpu.VMEM((N,), _f32),
            pltpu.VMEM((N,), _f32),
            pltpu.VMEM((N,), _f32),
        ],
    )
    return f(src2d, dst2d, esc0, esc1, ssrc, sdst)


def _sc_scalar1_body(src_hbm, dst_hbm, esc_hbm, ssrc_hbm, sdst_hbm,
                     ex_hbm, den_hbm,
                     ssrc_v, sdst_v, src_v, dst_v, esc_v, ex_v, den_v):
    cid = lax.axis_index("c")
    sid = lax.axis_index("s")
    wid = sid * NC + cid

    pltpu.sync_copy(ssrc_hbm, ssrc_v)
    pltpu.sync_copy(sdst_hbm, sdst_v)
    pltpu.sync_copy(src_hbm.at[wid], src_v)
    pltpu.sync_copy(dst_hbm.at[wid], dst_v)
    pltpu.sync_copy(esc_hbm.at[wid], esc_v)

    zeros16 = jnp.zeros((LANES,), _f32)

    @pl.loop(0, N // LANES)
    def _(i):
        den_v[pl.ds(i * LANES, LANES)] = zeros16

    @pl.loop(0, EPW // LANES)
    def _(g):
        sl = pl.ds(g * LANES, LANES)
        a = (plsc.load_gather(ssrc_v, [src_v[sl]])
             + plsc.load_gather(sdst_v, [dst_v[sl]]) + esc_v[sl])
        a = jnp.where(a > 0.0, a, 0.2 * a)
        ex = jnp.exp(a)
        ex_v[sl] = ex
        plsc.addupdate_scatter(den_v, [dst_v[sl]], ex)

    pltpu.sync_copy(ex_v, ex_hbm.at[wid])
    pltpu.sync_copy(den_v, den_hbm.at[wid])


def _sc_scalar1_pass(src2d, dst2d, esc, ssrc, sdst):
    f = pl.kernel(
        _sc_scalar1_body,
        out_type=(
            jax.ShapeDtypeStruct((NW, EPW), _f32),
            jax.ShapeDtypeStruct((NW, N), _f32),
        ),
        mesh=_sc_mesh(),
        compiler_params=_SC_PARAMS,
        scratch_types=[
            pltpu.VMEM((N,), _f32),
            pltpu.VMEM((N,), _f32),
            pltpu.VMEM((EPW,), jnp.int32),
            pltpu.VMEM((EPW,), jnp.int32),
            pltpu.VMEM((EPW,), _f32),
            pltpu.VMEM((EPW,), _f32),
            pltpu.VMEM((N,), _f32),
        ],
    )
    return f(src2d, dst2d, esc, ssrc, sdst)


def _sc_edge_body(h_hbm, src_hbm, dst_hbm, ex_hbm, num_hbm,
                  src_v, dst_v, ex_v, rows_a, rows_b, acc_sh, sem_a, sem_b):
    cid = lax.axis_index("c")
    sid = lax.axis_index("s")
    wid = sid * NC + cid

    zeros16 = jnp.zeros((LANES,), _f32)

    @pl.loop(0, CHUNK)
    def _(r):
        for k in range(HID // LANES):
            rows_a[r, pl.ds(k * LANES, LANES)] = zeros16

    @pl.loop(0, RPS // CHUNK)
    def _(z):
        pltpu.sync_copy(rows_a,
                        acc_sh.at[pl.ds(sid * RPS + z * CHUNK, CHUNK)])
    rem = RPS - (RPS // CHUNK) * CHUNK
    pltpu.sync_copy(rows_a.at[pl.ds(0, rem)],
                    acc_sh.at[pl.ds(sid * RPS + RPS - rem, rem)])
    plsc.subcore_barrier()

    def start_gather(j, buf, sem):
        return pltpu.async_copy(
            h_hbm.at[src_v.at[pl.ds(j * CHUNK, CHUNK)]], buf, sem)

    def wait_gather(j, buf, sem):
        pltpu.make_async_copy(
            h_hbm.at[src_v.at[pl.ds(j * CHUNK, CHUNK)]], buf, sem).wait()

    def scale_scatter(j, buf):
        off = j * CHUNK

        @pl.loop(0, CHUNK)
        def _(r):
            exs = ex_v[pl.ds(off + r, LANES)][0]
            for k in range(HID // LANES):
                rsl = pl.ds(k * LANES, LANES)
                buf[r, rsl] = buf[r, rsl] * exs

        pltpu.sync_copy(buf, acc_sh.at[dst_v.at[j]], add=True)

    @pl.loop(0, NSEG)
    def _(s):
        pltpu.sync_copy(src_hbm.at[wid].at[s], src_v)
        pltpu.sync_copy(dst_hbm.at[wid].at[s], dst_v)
        pltpu.sync_copy(ex_hbm.at[wid].at[s], ex_v.at[pl.ds(0, SEG)])

        start_gather(0, rows_a, sem_a)

        @pl.loop(0, (NCHUNK - 1) // 2)
        def _(t):
            j0 = 2 * t
            start_gather(j0 + 1, rows_b, sem_b)
            wait_gather(j0, rows_a, sem_a)
            scale_scatter(j0, rows_a)
            start_gather(j0 + 2, rows_a, sem_a)
            wait_gather(j0 + 1, rows_b, sem_b)
            scale_scatter(j0 + 1, rows_b)

        wait_gather(NCHUNK - 1, rows_a, sem_a)
        scale_scatter(NCHUNK - 1, rows_a)

    plsc.subcore_barrier()

    pltpu.sync_copy(acc_sh.at[pl.ds(sid * RPS, RPS)],
                    num_hbm.at[cid].at[pl.ds(sid * RPS, RPS)])


def _sc_edge_pass(h, src3d, dst4d, ex3d):
    f = pl.kernel(
        _sc_edge_body,
        out_type=jax.ShapeDtypeStruct((NC, N, HID), _f32),
        mesh=_sc_mesh(),
        compiler_params=_SC_PARAMS,
        scratch_types=[
            pltpu.VMEM((SEG,), jnp.int32),
            pltpu.VMEM((NCHUNK, CHUNK), jnp.int32),
            pltpu.VMEM((SEG + LANES,), _f32),
            pltpu.VMEM((CHUNK, HID), _f32),
            pltpu.VMEM((CHUNK, HID), _f32),
            pltpu.VMEM_SHARED((N, HID), _f32),
            pltpu.SemaphoreType.DMA,
            pltpu.SemaphoreType.DMA,
        ],
    )
    return f(h, src3d, dst4d, ex3d)


def _mid_body(nump_ref, denp_ref, cntp_ref, esump_ref, h0_ref, ss0_ref,
              sd0_ref, b0_ref, w1_ref, as1_ref, ad1_ref,
              h1_ref, ss1_ref, sd1_ref, cnt_ref):
    den = jnp.sum(denp_ref[...], axis=0)
    cnt = jnp.sum(cntp_ref[...], axis=0)
    esum = jnp.sum(esump_ref[...], axis=0)
    num = nump_ref[0] + nump_ref[1]
    a_self = ss0_ref[...] + sd0_ref[...] + esum / jnp.maximum(cnt, 1.0)
    a_self = jnp.where(a_self > 0.0, a_self, 0.2 * a_self)
    exs = jnp.exp(a_self)
    h0 = h0_ref[...]
    num = num + exs[:, None] * h0
    den = den + exs
    x1 = num / (den + 1e-16)[:, None] + b0_ref[...][None, :]
    x1 = jnp.maximum(x1, 0.0)
    h1 = jnp.dot(x1, w1_ref[...], preferred_element_type=_f32)
    h1_ref[...] = h1
    ss1_ref[...] = jnp.sum(h1 * as1_ref[...][None, :], axis=1)
    sd1_ref[...] = jnp.sum(h1 * ad1_ref[...][None, :], axis=1)
    cnt_ref[...] = cnt


def _mid_tc(num_p, den_p, cnt_p, esum_p, h0, ss0, sd0, b0, W1, as1, ad1):
    return pl.pallas_call(
        _mid_body,
        out_shape=(
            jax.ShapeDtypeStruct((N, HID), _f32),
            jax.ShapeDtypeStruct((N,), _f32),
            jax.ShapeDtypeStruct((N,), _f32),
            jax.ShapeDtypeStruct((N,), _f32),
        ),
    )(num_p, den_p, cnt_p, esum_p, h0, ss0, sd0, b0, W1, as1, ad1)


def _final_body(nump_ref, denp_ref, esump_ref, cnt_ref, h1_ref, ss1_ref,
                sd1_ref, b1_ref, batch_ref, gw_ref, gb_ref, lw_ref, lb_ref,
                out_ref):
    den = jnp.sum(denp_ref[...], axis=0)
    esum = jnp.sum(esump_ref[...], axis=0)
    num = nump_ref[0] + nump_ref[1]
    a_self = ss1_ref[...] + sd1_ref[...] + esum / jnp.maximum(cnt_ref[...], 1.0)
    a_self = jnp.where(a_self > 0.0, a_self, 0.2 * a_self)
    exs = jnp.exp(a_self)
    h1 = h1_ref[...]
    num = num + exs[:, None] * h1
    den = den + exs
    h2 = num / (den + 1e-16)[:, None] + b1_ref[...][None, :]

    gate = jnp.sum(h2 * gw_ref[...][:, 0][None, :], axis=1) + gb_ref[0]
    ids = lax.broadcasted_iota(jnp.int32, (G, N), 0)
    msk = batch_ref[...][None, :] == ids
    m = jnp.max(jnp.where(msk, gate[None, :], -1e30), axis=1)
    mb = jnp.sum(jnp.where(msk, m[:, None], 0.0), axis=0)
    ex = jnp.exp(gate - mb)
    den_g = jnp.sum(jnp.where(msk, ex[None, :], 0.0), axis=1)
    denb = jnp.sum(jnp.where(msk, den_g[:, None], 0.0), axis=0)
    coef = ex / (denb + 1e-16)
    pool_w = jnp.where(msk, coef[None, :], 0.0)
    pooled = jnp.dot(pool_w, h2, preferred_element_type=_f32)
    out_ref[...] = (jnp.dot(pooled, lw_ref[...], preferred_element_type=_f32)
                    + lb_ref[...][None, :])


def _final_tc(num_p, den_p, esum_p, cnt, h1, ss1, sd1, b1, batch, gw, gb,
              lw, lb):
    return pl.pallas_call(
        _final_body,
        out_shape=jax.ShapeDtypeStruct((G, HID), _f32),
    )(num_p, den_p, esum_p, cnt, h1, ss1, sd1, b1, batch, gw, gb, lw, lb)


def kernel(x, edge_index, edge_attr, batch, W0, att_src0, att_dst0,
           att_edge0, We0, b0, W1, att_src1, att_dst1, att_edge1, We1, b1,
           gate_w, gate_b, lin1_w, lin1_b):
    src2d = edge_index[0].reshape(NW, EPW)
    dst2d = edge_index[1].reshape(NW, EPW)
    src3d = edge_index[0].reshape(NW, NSEG, SEG)
    dst4d = edge_index[1].reshape(NW, NSEG, NCHUNK, CHUNK)

    h0, ss0, sd0 = _prep_tc(x, W0, att_src0, att_dst0)
    ea8 = edge_attr.reshape(E // _EPR, HID)
    e0p, e1p = _escore_tc(ea8, We0, att_edge0, We1, att_edge1)
    esc0 = e0p.reshape(NW, EPW)
    esc1 = e1p.reshape(NW, EPW)

    ex0, den0_p, cnt_p, esum0_p, esum1_p = _sc_scalar0_pass(
        src2d, dst2d, esc0, esc1, ss0, sd0)
    num0 = _sc_edge_pass(h0, src3d, dst4d, ex0.reshape(NW, NSEG, SEG))
    h1, ss1, sd1, cnt = _mid_tc(num0, den0_p, cnt_p, esum0_p, h0, ss0, sd0,
                                b0, W1, att_src1, att_dst1)
    ex1, den1_p = _sc_scalar1_pass(src2d, dst2d, esc1, ss1, sd1)
    num1 = _sc_edge_pass(h1, src3d, dst4d, ex1.reshape(NW, NSEG, SEG))
    return _final_tc(num1, den1_p, esum1_p, cnt, h1, ss1, sd1, b1, batch,
                     gate_w, gate_b, lin1_w, lin1_b)
